# baseline (speedup 1.0000x reference)
import jax
import jax.numpy as jnp
from jax.experimental import pallas as pl
from jax.experimental.pallas import tpu as pltpu


def _split_body(x_ref, o1_ref, o2_ref, sem1, sem2):
    nt = o1_ref.shape[1]
    c1 = pltpu.make_async_copy(x_ref.at[:, :nt, :], o1_ref, sem1)
    c2 = pltpu.make_async_copy(x_ref.at[:, nt:, :], o2_ref, sem2)
    c1.start()
    c2.start()
    c1.wait()
    c2.wait()


def kernel(x):
    B, C, H, W = x.shape
    HW = H * W
    nt = HW // 2
    y = jnp.transpose(x, (0, 2, 3, 1)).reshape(B, HW, C)
    o1, o2 = pl.pallas_call(
        _split_body,
        in_specs=[pl.BlockSpec(memory_space=pl.ANY)],
        out_specs=[
            pl.BlockSpec(memory_space=pl.ANY),
            pl.BlockSpec(memory_space=pl.ANY),
        ],
        out_shape=[
            jax.ShapeDtypeStruct((B, nt, C), x.dtype),
            jax.ShapeDtypeStruct((B, nt, C), x.dtype),
        ],
        scratch_shapes=[pltpu.SemaphoreType.DMA, pltpu.SemaphoreType.DMA],
    )(y)
    X1 = o1.reshape(B, H, nt // W, C).transpose(0, 3, 1, 2)
    X2 = o2.reshape(B, H, nt // W, C).transpose(0, 3, 1, 2)
    return (X1, X2)


# speedup vs baseline: 33.4430x; 33.4430x over previous
import jax
import jax.numpy as jnp
from jax.experimental import pallas as pl
from jax.experimental.pallas import tpu as pltpu


def _split_body(x_ref, o1_ref, o2_ref):
    j = pl.program_id(1)

    @pl.when(j == 0)
    def _():
        o1_ref[...] = x_ref[...]

    @pl.when(j == 1)
    def _():
        o2_ref[...] = x_ref[...]


def kernel(x):
    B, C, H, W = x.shape
    HW = H * W
    nt = HW // 2
    y = jnp.transpose(x, (0, 2, 3, 1)).reshape(B, HW, C)
    o1, o2 = pl.pallas_call(
        _split_body,
        grid=(B, 2),
        in_specs=[pl.BlockSpec((1, nt, C), lambda i, j: (i, j, 0))],
        out_specs=[
            pl.BlockSpec((1, nt, C), lambda i, j: (i, 0, 0)),
            pl.BlockSpec((1, nt, C), lambda i, j: (i, 0, 0)),
        ],
        out_shape=[
            jax.ShapeDtypeStruct((B, nt, C), x.dtype),
            jax.ShapeDtypeStruct((B, nt, C), x.dtype),
        ],
    )(y)
    X1 = o1.reshape(B, H, nt // W, C).transpose(0, 3, 1, 2)
    X2 = o2.reshape(B, H, nt // W, C).transpose(0, 3, 1, 2)
    return (X1, X2)


# speedup vs baseline: 44.8618x; 1.3414x over previous
import jax
import jax.numpy as jnp
from jax.experimental import pallas as pl
from jax.experimental.pallas import tpu as pltpu


def _split_body(x_ref, o1_ref, o2_ref):
    nt = o1_ref.shape[1]
    o1_ref[...] = x_ref[:, :nt, :]
    o2_ref[...] = x_ref[:, nt:, :]


def kernel(x):
    B, C, H, W = x.shape
    HW = H * W
    nt = HW // 2
    y = jnp.transpose(x, (0, 2, 3, 1)).reshape(B, HW, C)
    o1, o2 = pl.pallas_call(
        _split_body,
        grid=(B // 2,),
        in_specs=[pl.BlockSpec((2, HW, C), lambda i: (i, 0, 0))],
        out_specs=[
            pl.BlockSpec((2, nt, C), lambda i: (i, 0, 0)),
            pl.BlockSpec((2, nt, C), lambda i: (i, 0, 0)),
        ],
        out_shape=[
            jax.ShapeDtypeStruct((B, nt, C), x.dtype),
            jax.ShapeDtypeStruct((B, nt, C), x.dtype),
        ],
    )(y)
    X1 = o1.reshape(B, H, nt // W, C).transpose(0, 3, 1, 2)
    X2 = o2.reshape(B, H, nt // W, C).transpose(0, 3, 1, 2)
    return (X1, X2)


# speedup vs baseline: 47.2601x; 1.0535x over previous
import jax
import jax.numpy as jnp
from jax.experimental import pallas as pl
from jax.experimental.pallas import tpu as pltpu


def _split_body(x_ref, o1_ref, o2_ref):
    nt = o1_ref.shape[1]
    o1_ref[...] = x_ref[:, :nt, :]
    o2_ref[...] = x_ref[:, nt:, :]


def kernel(x):
    B, C, H, W = x.shape
    HW = H * W
    nt = HW // 2
    y = jnp.transpose(x, (0, 2, 3, 1)).reshape(B, HW, C)
    o1, o2 = pl.pallas_call(
        _split_body,
        grid=(B // 4,),
        in_specs=[pl.BlockSpec((4, HW, C), lambda i: (i, 0, 0))],
        out_specs=[
            pl.BlockSpec((4, nt, C), lambda i: (i, 0, 0)),
            pl.BlockSpec((4, nt, C), lambda i: (i, 0, 0)),
        ],
        out_shape=[
            jax.ShapeDtypeStruct((B, nt, C), x.dtype),
            jax.ShapeDtypeStruct((B, nt, C), x.dtype),
        ],
    )(y)
    X1 = o1.reshape(B, H, nt // W, C).transpose(0, 3, 1, 2)
    X2 = o2.reshape(B, H, nt // W, C).transpose(0, 3, 1, 2)
    return (X1, X2)
